# baseline (speedup 1.0000x reference)
import functools

import jax
import jax.numpy as jnp
from jax import lax
from jax.experimental import pallas as pl
from jax.experimental.pallas import tpu as pltpu
from jax.experimental.pallas import tpu_sc as plsc

NUM_SPEAKER = 100000
EMB_DIM = 64
BATCH = 4096

_info = plsc.get_sparse_core_info()
_NC, _NS = _info.num_cores, _info.num_subcores
_NW = _NC * _NS
_B_PER_W = BATCH // _NW


@functools.partial(
    pl.kernel,
    mesh=plsc.VectorSubcoreMesh(core_axis_name="c", subcore_axis_name="s"),
    out_type=jax.ShapeDtypeStruct((BATCH, EMB_DIM), jnp.float32),
    scratch_types=[
        pltpu.VMEM((_B_PER_W,), jnp.int32),
        pltpu.VMEM((_B_PER_W, EMB_DIM), jnp.float32),
        pltpu.SemaphoreType.DMA,
    ],
    compiler_params=pltpu.CompilerParams(use_tc_tiling_on_sc=False),
)
def _gather_kernel(table_hbm, idx_hbm, out_hbm, idx_v, rows_v, sem):
    wid = lax.axis_index("s") * _NC + lax.axis_index("c")
    base = wid * _B_PER_W
    pltpu.sync_copy(idx_hbm.at[pl.ds(base, _B_PER_W)], idx_v)
    pltpu.async_copy(table_hbm.at[idx_v], rows_v, sem).wait()
    pltpu.sync_copy(rows_v, out_hbm.at[pl.ds(base, _B_PER_W)])


def kernel(sid, cropped_waveform, emb_table):
    del cropped_waveform
    return _gather_kernel(emb_table, sid.astype(jnp.int32))


# speedup vs baseline: 1.0598x; 1.0598x over previous
import functools

import jax
import jax.numpy as jnp
from jax import lax
from jax.experimental import pallas as pl
from jax.experimental.pallas import tpu as pltpu
from jax.experimental.pallas import tpu_sc as plsc

NUM_SPEAKER = 100000
EMB_DIM = 64
BATCH = 4096
LANES = 128

_info = plsc.get_sparse_core_info()
_NC, _NS = _info.num_cores, _info.num_subcores
_NW = _NC * _NS
_B_PER_W = BATCH // _NW


@functools.partial(
    pl.kernel,
    mesh=plsc.VectorSubcoreMesh(core_axis_name="c", subcore_axis_name="s"),
    out_type=jax.ShapeDtypeStruct((EMB_DIM, BATCH), jnp.float32),
    scratch_types=[
        pltpu.VMEM((_B_PER_W,), jnp.int32),
        pltpu.VMEM((_B_PER_W, LANES), jnp.float32),
        pltpu.VMEM((EMB_DIM, _B_PER_W), jnp.float32),
        pltpu.SemaphoreType.DMA,
    ],
    compiler_params=pltpu.CompilerParams(needs_layout_passes=False),
)
def _gather_kernel(table_pad, idx_hbm, out_t, idx_v, rows_v, block_t, sem):
    wid = lax.axis_index("s") * _NC + lax.axis_index("c")
    base = pl.multiple_of(wid * _B_PER_W, _B_PER_W)
    pltpu.sync_copy(idx_hbm.at[pl.ds(base, _B_PER_W)], idx_v)
    pltpu.async_copy(table_pad.at[idx_v], rows_v, sem).wait()

    lane_iota = lax.iota(jnp.int32, 16)

    def transpose_row(e, carry):
        e_vec = lane_iota * 0 + e
        for ib in range(_B_PER_W // 16):
            rows = lane_iota + ib * 16
            vals = plsc.load_gather(rows_v, [rows, e_vec])
            block_t[e, pl.ds(ib * 16, 16)] = vals
        return carry

    lax.fori_loop(0, EMB_DIM, transpose_row, 0, unroll=False)

    pltpu.sync_copy(block_t, out_t.at[:, pl.ds(base, _B_PER_W)])


def kernel(sid, cropped_waveform, emb_table):
    del cropped_waveform
    table_pad = jnp.pad(emb_table, ((0, 0), (0, LANES - EMB_DIM)))
    out_t = _gather_kernel(table_pad, sid.astype(jnp.int32))
    return out_t.T


# speedup vs baseline: 2.4826x; 2.3425x over previous
import functools

import jax
import jax.numpy as jnp
from jax import lax
from jax.experimental import pallas as pl
from jax.experimental.pallas import tpu as pltpu
from jax.experimental.pallas import tpu_sc as plsc

NUM_SPEAKER = 100000
EMB_DIM = 64
BATCH = 4096

_info = plsc.get_sparse_core_info()
_NC, _NS = _info.num_cores, _info.num_subcores
_NW = _NC * _NS
_ROWS_PER_W = EMB_DIM // _NW


@functools.partial(
    pl.kernel,
    mesh=plsc.VectorSubcoreMesh(core_axis_name="c", subcore_axis_name="s"),
    out_type=jax.ShapeDtypeStruct((EMB_DIM, BATCH), jnp.float32),
    scratch_types=[
        pltpu.VMEM((NUM_SPEAKER,), jnp.float32),
        pltpu.VMEM((BATCH,), jnp.int32),
        pltpu.VMEM((BATCH,), jnp.float32),
    ],
    compiler_params=pltpu.CompilerParams(
        use_tc_tiling_on_sc=True, needs_layout_passes=False
    ),
)
def _lane_gather_kernel(tab_t, sid_hbm, out_t, row_v, sid_v, out_v):
    wid = lax.axis_index("s") * _NC + lax.axis_index("c")
    pltpu.sync_copy(sid_hbm, sid_v)
    for r in range(_ROWS_PER_W):
        c = wid * _ROWS_PER_W + r
        pltpu.sync_copy(tab_t.at[c], row_v)

        def body(i, carry):
            off = pl.multiple_of(i * 16, 16)
            idx = sid_v[pl.ds(off, 16)]
            out_v[pl.ds(off, 16)] = plsc.load_gather(row_v, [idx])
            return carry

        lax.fori_loop(0, BATCH // 16, body, 0, unroll=False)
        pltpu.sync_copy(out_v, out_t.at[c])


def kernel(sid, cropped_waveform, emb_table):
    del cropped_waveform
    out_t = _lane_gather_kernel(emb_table.T, sid.astype(jnp.int32))
    return out_t.T
